# baseline (speedup 1.0000x reference)
import jax
import jax.numpy as jnp
from jax.experimental import pallas as pl

N = 10000
K = 32
R = 3
EPS = 1e-8


def _gcn_conv(x, src, dst, W, b):
    n = x.shape[0]
    xw = x @ W
    loops = jnp.arange(n, dtype=src.dtype)
    s = jnp.concatenate([src, loops])
    d = jnp.concatenate([dst, loops])
    deg = jax.ops.segment_sum(jnp.ones(s.shape[0], dtype=xw.dtype), d, num_segments=n)
    dinv = jax.lax.rsqrt(jnp.maximum(deg, 1e-12))
    norm = dinv[s] * dinv[d]
    out = jax.ops.segment_sum(xw[s] * norm[:, None], d, num_segments=n)
    return out + b


def _batchnorm(x, gamma, beta):
    mu = jnp.mean(x, axis=0)
    var = jnp.var(x, axis=0)
    return gamma * (x - mu) / jnp.sqrt(var + 1e-5) + beta


def _residual_vq(x, codebooks):
    residual = x
    commit = jnp.float32(0.0)
    ids = []
    for l in range(R):
        cb = codebooks[l]
        rn = residual / (jnp.linalg.norm(residual, axis=-1, keepdims=True) + EPS)
        cbn = cb / (jnp.linalg.norm(cb, axis=-1, keepdims=True) + EPS)
        sim = rn @ cbn.T
        idx = jnp.argmax(sim, axis=-1)
        q = jnp.take(cb, idx, axis=0)
        commit = commit + 0.25 * jnp.mean((q - residual) ** 2)
        residual = residual - q
        ids.append(idx)
    return jnp.stack(ids, axis=1), commit


def _head_kernel(h_ref, wl_ref, bl_ref, o_ref):
    o_ref[...] = jnp.dot(h_ref[...], wl_ref[...],
                         preferred_element_type=jnp.float32) + bl_ref[...]


def kernel(x, edge_index, W1, b1, W2, b2, gamma, beta, cb1, cb2, Wl, bl, Wg, bg):
    src = edge_index[0]
    dst = edge_index[1]
    h = _gcn_conv(x, src, dst, W1, b1)
    h = _batchnorm(h, gamma, beta)
    h = jax.nn.relu(h)
    ids1, c1 = _residual_vq(h, cb1)
    h = _gcn_conv(h, src, dst, W2, b2)
    ids2, c2 = _residual_vq(h, cb2)
    total_commit = c1 + c2
    id_concat = jnp.concatenate([ids1, ids2], axis=1)
    o1 = pl.pallas_call(
        _head_kernel,
        out_shape=jax.ShapeDtypeStruct((N, Wl.shape[1]), jnp.float32),
    )(h, Wl, bl[None, :])
    return (o1, total_commit, id_concat, h @ Wg + bg)


# speedup vs baseline: 6.4810x; 6.4810x over previous
import functools

import jax
import jax.numpy as jnp
from jax import lax
from jax.experimental import pallas as pl
from jax.experimental.pallas import tpu as pltpu
from jax.experimental.pallas import tpu_sc as plsc

N = 10000
D = 128
E = 320000
K = 32
R = 3
EPS = 1e-8

NC = 2
NS = 16
NW = NC * NS
NP = 10240
ROWS_PER_TILE = NP // NS
C = 128
T = 80
EP = NW * T * C
DW = 16

_mesh = plsc.VectorSubcoreMesh(core_axis_name="c", subcore_axis_name="s")


@functools.partial(
    pl.kernel,
    out_type=jax.ShapeDtypeStruct((NC, NP), jnp.float32),
    mesh=_mesh,
    scratch_types=[
        pltpu.VMEM((C,), jnp.int32),
        pltpu.VMEM((C,), jnp.float32),
        pltpu.VMEM((ROWS_PER_TILE,), jnp.float32),
        pltpu.VMEM_SHARED((NP,), jnp.float32),
    ],
)
def _deg_kernel(dst_hbm, out_hbm, didx_v, ones_v, stage_v, acc_sh):
    c = lax.axis_index("c")
    s = lax.axis_index("s")
    wid = s * NC + c

    def fill(i, _):
        ones_v[pl.ds(i * 16, 16)] = jnp.full((16,), 1.0, jnp.float32)
        return 0

    lax.fori_loop(0, C // 16, fill, 0)

    def zfill(i, _):
        stage_v[pl.ds(i * 16, 16)] = jnp.zeros((16,), jnp.float32)
        return 0

    lax.fori_loop(0, ROWS_PER_TILE // 16, zfill, 0)

    base = s * ROWS_PER_TILE
    pltpu.sync_copy(stage_v, acc_sh.at[pl.ds(base, ROWS_PER_TILE)])
    plsc.subcore_barrier()

    def body(j, _):
        pltpu.sync_copy(dst_hbm.at[pl.ds(wid * T * C + j * C, C)], didx_v)
        pltpu.sync_copy(ones_v, acc_sh.at[didx_v], add=True)
        return 0

    lax.fori_loop(0, T, body, 0)
    plsc.subcore_barrier()

    pltpu.sync_copy(acc_sh.at[pl.ds(base, ROWS_PER_TILE)], stage_v)
    pltpu.sync_copy(stage_v, out_hbm.at[c, pl.ds(base, ROWS_PER_TILE)])


@functools.partial(
    pl.kernel,
    out_type=jax.ShapeDtypeStruct((NC, NP, D), jnp.float32),
    mesh=_mesh,
    scratch_types=[
        pltpu.VMEM((C,), jnp.int32),
        pltpu.VMEM((C,), jnp.int32),
        pltpu.VMEM((C, D), jnp.float32),
        pltpu.VMEM_SHARED((NP, D), jnp.float32),
        pltpu.SemaphoreType.DMA,
    ],
)
def _mp_kernel(y_hbm, src_hbm, dst_hbm, out_hbm, sidx_v, didx_v, rows_v, acc_sh, sem):
    c = lax.axis_index("c")
    s = lax.axis_index("s")
    wid = s * NC + c

    def zero_rows(i, _):
        r = i // (D // 16)
        q = i % (D // 16)
        rows_v[r, pl.ds(q * 16, 16)] = jnp.zeros((16,), jnp.float32)
        return 0

    lax.fori_loop(0, C * (D // 16), zero_rows, 0)

    base = s * ROWS_PER_TILE
    for k in range(ROWS_PER_TILE // C):
        pltpu.sync_copy(rows_v, acc_sh.at[pl.ds(base + k * C, C)])
    plsc.subcore_barrier()

    def body(j, _):
        e = wid * T * C + j * C
        pltpu.sync_copy(src_hbm.at[pl.ds(e, C)], sidx_v)
        pltpu.sync_copy(dst_hbm.at[pl.ds(e, C)], didx_v)
        pltpu.async_copy(y_hbm.at[sidx_v], rows_v, sem).wait()
        pltpu.sync_copy(rows_v, acc_sh.at[didx_v], add=True)
        return 0

    lax.fori_loop(0, T, body, 0)
    plsc.subcore_barrier()

    for k in range(ROWS_PER_TILE // C):
        pltpu.sync_copy(acc_sh.at[pl.ds(base + k * C, C)], rows_v)
        pltpu.sync_copy(rows_v, out_hbm.at[c, pl.ds(base + k * C, C)])


def _residual_vq(x, codebooks):
    residual = x
    commit = jnp.float32(0.0)
    ids = []
    for l in range(R):
        cb = codebooks[l]
        rn = residual / (jnp.linalg.norm(residual, axis=-1, keepdims=True) + EPS)
        cbn = cb / (jnp.linalg.norm(cb, axis=-1, keepdims=True) + EPS)
        sim = rn @ cbn.T
        idx = jnp.argmax(sim, axis=-1)
        q = jnp.take(cb, idx, axis=0)
        commit = commit + 0.25 * jnp.mean((q - residual) ** 2)
        residual = residual - q
        ids.append(idx)
    return jnp.stack(ids, axis=1), commit


def kernel(x, edge_index, W1, b1, W2, b2, gamma, beta, cb1, cb2, Wl, bl, Wg, bg):
    src = edge_index[0]
    dst = edge_index[1]
    pad = jnp.full((EP - E,), NP - 1, dtype=jnp.int32)
    src2d = jnp.concatenate([src, pad])
    dst2d = jnp.concatenate([dst, pad])

    degp = _deg_kernel(dst2d)
    deg = degp[0, :N] + degp[1, :N] + 1.0
    dinv = lax.rsqrt(jnp.maximum(deg, 1e-12))[:, None]

    def conv(h, W, b):
        y = dinv * (h @ W)
        y_pad = jnp.concatenate([y, jnp.zeros((NP - N, D), jnp.float32)], axis=0)
        p = _mp_kernel(y_pad, src2d, dst2d)
        return dinv * (p[0, :N] + p[1, :N] + y) + b

    h = conv(x, W1, b1)
    mu = jnp.mean(h, axis=0)
    var = jnp.var(h, axis=0)
    h = gamma * (h - mu) / jnp.sqrt(var + 1e-5) + beta
    h = jax.nn.relu(h)
    ids1, c1 = _residual_vq(h, cb1)
    h = conv(h, W2, b2)
    ids2, c2 = _residual_vq(h, cb2)
    total_commit = c1 + c2
    id_concat = jnp.concatenate([ids1, ids2], axis=1)
    return (h @ Wl + bl, total_commit, id_concat, h @ Wg + bg)


# speedup vs baseline: 6.8115x; 1.0510x over previous
import functools

import jax
import jax.numpy as jnp
from jax import lax
from jax.experimental import pallas as pl
from jax.experimental.pallas import tpu as pltpu
from jax.experimental.pallas import tpu_sc as plsc

N = 10000
D = 128
E = 320000
K = 32
R = 3
EPS = 1e-8

NC = 2
NS = 16
NW = NC * NS
NP = 10240
ROWS_PER_TILE = NP // NS
C = 128
T = 80
EP = NW * T * C
DW = 16


def _deg_body(dst_hbm, out_hbm, didx_v, ones_v, stage_v, acc_sh):
    c = lax.axis_index("c")
    s = lax.axis_index("s")
    wid = s * NC + c

    def fill(i, _):
        ones_v[pl.ds(i * 16, 16)] = jnp.full((16,), 1.0, jnp.float32)
        return 0

    lax.fori_loop(0, C // 16, fill, 0)

    def zfill(i, _):
        stage_v[pl.ds(i * 16, 16)] = jnp.zeros((16,), jnp.float32)
        return 0

    lax.fori_loop(0, ROWS_PER_TILE // 16, zfill, 0)

    base = s * ROWS_PER_TILE
    pltpu.sync_copy(stage_v, acc_sh.at[pl.ds(base, ROWS_PER_TILE)])
    plsc.subcore_barrier()

    def body(j, _):
        pltpu.sync_copy(dst_hbm.at[pl.ds(wid * T * C + j * C, C)], didx_v)
        pltpu.sync_copy(ones_v, acc_sh.at[didx_v], add=True)
        return 0

    lax.fori_loop(0, T, body, 0)
    plsc.subcore_barrier()

    pltpu.sync_copy(acc_sh.at[pl.ds(base, ROWS_PER_TILE)], stage_v)
    pltpu.sync_copy(stage_v, out_hbm.at[c, pl.ds(base, ROWS_PER_TILE)])


def _mp_body(y_hbm, src_hbm, dst_hbm, out_hbm, sidx_v, didx_v, rows_v, acc_sh, sem):
    c = lax.axis_index("c")
    s = lax.axis_index("s")
    wid = s * NC + c

    def zero_rows(i, _):
        r = i // (D // 16)
        q = i % (D // 16)
        rows_v[r, pl.ds(q * 16, 16)] = jnp.zeros((16,), jnp.float32)
        return 0

    lax.fori_loop(0, C * (D // 16), zero_rows, 0)

    base = s * ROWS_PER_TILE
    for k in range(ROWS_PER_TILE // C):
        pltpu.sync_copy(rows_v, acc_sh.at[pl.ds(base + k * C, C)])
    plsc.subcore_barrier()

    def body(j, _):
        e = wid * T * C + j * C
        pltpu.sync_copy(src_hbm.at[pl.ds(e, C)], sidx_v)
        pltpu.sync_copy(dst_hbm.at[pl.ds(e, C)], didx_v)
        pltpu.async_copy(y_hbm.at[sidx_v], rows_v, sem).wait()
        pltpu.sync_copy(rows_v, acc_sh.at[didx_v], add=True)
        return 0

    lax.fori_loop(0, T, body, 0)
    plsc.subcore_barrier()

    for k in range(ROWS_PER_TILE // C):
        pltpu.sync_copy(acc_sh.at[pl.ds(base + k * C, C)], rows_v)
        pltpu.sync_copy(rows_v, out_hbm.at[c, pl.ds(base + k * C, C)])


def _tc_pre_body(x_ref, w1_ref, degp_ref, y1_ref, dinv_ref):
    dp = degp_ref[...]
    deg = dp[0] + dp[1] + 1.0
    dinv = lax.rsqrt(jnp.maximum(deg, 1e-12))
    dinv_ref[...] = dinv
    xw = jnp.dot(x_ref[...], w1_ref[...], preferred_element_type=jnp.float32)
    y1_ref[0:N, :] = dinv[0:N] * xw
    y1_ref[N:NP, :] = jnp.zeros((NP - N, D), jnp.float32)


def _tc_mid_body(p_ref, y1_ref, dinv_ref, b1_ref, gamma_ref, beta_ref, w2_ref,
                 h1_ref, y2_ref):
    p = p_ref[...]
    dinv = dinv_ref[0:N]
    h = dinv * (p[0, 0:N] + p[1, 0:N] + y1_ref[0:N]) + b1_ref[...]
    mu = jnp.mean(h, axis=0, keepdims=True)
    var = jnp.mean((h - mu) ** 2, axis=0, keepdims=True)
    h = gamma_ref[...] * (h - mu) / jnp.sqrt(var + 1e-5) + beta_ref[...]
    h = jnp.maximum(h, 0.0)
    h1_ref[...] = h
    xw2 = jnp.dot(h, w2_ref[...], preferred_element_type=jnp.float32)
    y2_ref[0:N, :] = dinv * xw2
    y2_ref[N:NP, :] = jnp.zeros((NP - N, D), jnp.float32)


VB = 2000
VG = N // VB


def _vq(h, cb_all):
    residual = h
    b = h.shape[0]
    commit = jnp.float32(0.0)
    packed = jnp.zeros((b, 1), jnp.int32)
    for l in range(R):
        cb = cb_all[l]
        rn = residual / (jnp.sqrt(jnp.sum(residual * residual, axis=-1,
                                          keepdims=True)) + EPS)
        cbn = cb / (jnp.sqrt(jnp.sum(cb * cb, axis=-1, keepdims=True)) + EPS)
        sim = lax.dot_general(rn, cbn, (((1,), (1,)), ((), ())),
                              preferred_element_type=jnp.float32)
        mx = jnp.max(sim, axis=-1, keepdims=True)
        lane = lax.broadcasted_iota(jnp.int32, (b, K), 1)
        idx = jnp.min(jnp.where(sim >= mx, lane, K), axis=-1, keepdims=True)
        q = jnp.zeros_like(residual)
        for k in range(K):
            q = jnp.where(idx == k, cb[k][None, :], q)
        commit = commit + 0.25 * (jnp.sum((q - residual) ** 2) / (N * D))
        residual = residual - q
        packed = packed + (idx << (5 * l))
    return packed, commit


def _tc_vq1_body(h1_ref, cb1_ref, ids_ref, c1_ref):
    i = pl.program_id(0)
    packed, commit = _vq(h1_ref[...], cb1_ref[...])
    ids_ref[...] = packed

    @pl.when(i == 0)
    def _():
        c1_ref[...] = jnp.zeros((1, 1), jnp.float32)

    c1_ref[...] += jnp.reshape(commit, (1, 1))


def _tc_fin_body(p_ref, y2_ref, dinv_ref, b2_ref, cb2_ref, wl_ref, bl_ref,
                 wg_ref, bg_ref, c1_ref, o1_ref, og_ref, ids_ref, cm_ref):
    i = pl.program_id(0)
    p = p_ref[...]
    h = dinv_ref[...] * (p[0] + p[1] + y2_ref[...]) + b2_ref[...]
    packed, c2 = _vq(h, cb2_ref[...])
    ids_ref[...] = packed

    @pl.when(i == 0)
    def _():
        cm_ref[...] = c1_ref[...]

    cm_ref[...] += jnp.reshape(c2, (1, 1))
    o1_ref[...] = jnp.dot(h, wl_ref[...], preferred_element_type=jnp.float32) \
        + bl_ref[...]
    og_ref[...] = jnp.dot(h, wg_ref[...], preferred_element_type=jnp.float32) \
        + bg_ref[...]


def _mk(body, out_shapes):
    return pl.pallas_call(body, out_shape=out_shapes)


_full = lambda shape: pl.BlockSpec(shape, lambda i: tuple(0 for _ in shape))

_tc_vq1_call = pl.pallas_call(
    _tc_vq1_body,
    grid=(VG,),
    in_specs=[
        pl.BlockSpec((VB, D), lambda i: (i, 0)),
        _full((R, K, D)),
    ],
    out_specs=[
        pl.BlockSpec((VB, 1), lambda i: (i, 0)),
        _full((1, 1)),
    ],
    out_shape=[
        jax.ShapeDtypeStruct((N, 1), jnp.int32),
        jax.ShapeDtypeStruct((1, 1), jnp.float32),
    ],
)


@functools.cache
def _sc_kernels():
    mesh = plsc.VectorSubcoreMesh(core_axis_name="c", subcore_axis_name="s",
                                  num_cores=NC, num_subcores=NS)
    deg = pl.kernel(
        _deg_body,
        out_type=jax.ShapeDtypeStruct((NC, NP), jnp.float32),
        mesh=mesh,
        scratch_types=[
            pltpu.VMEM((C,), jnp.int32),
            pltpu.VMEM((C,), jnp.float32),
            pltpu.VMEM((ROWS_PER_TILE,), jnp.float32),
            pltpu.VMEM_SHARED((NP,), jnp.float32),
        ],
    )
    mp = pl.kernel(
        _mp_body,
        out_type=jax.ShapeDtypeStruct((NC, NP, D), jnp.float32),
        mesh=mesh,
        scratch_types=[
            pltpu.VMEM((C,), jnp.int32),
            pltpu.VMEM((C,), jnp.int32),
            pltpu.VMEM((C, D), jnp.float32),
            pltpu.VMEM_SHARED((NP, D), jnp.float32),
            pltpu.SemaphoreType.DMA,
        ],
    )
    return deg, mp


_tc_pre = _mk(_tc_pre_body, [
    jax.ShapeDtypeStruct((NP, D), jnp.float32),
    jax.ShapeDtypeStruct((NP, 1), jnp.float32),
])
_tc_mid = _mk(_tc_mid_body, [
    jax.ShapeDtypeStruct((N, D), jnp.float32),
    jax.ShapeDtypeStruct((NP, D), jnp.float32),
])


def kernel(x, edge_index, W1, b1, W2, b2, gamma, beta, cb1, cb2, Wl, bl, Wg, bg):
    src = edge_index[0]
    dst = edge_index[1]
    pad = jnp.full((EP - E,), NP - 1, dtype=jnp.int32)
    srcp = jnp.concatenate([src, pad])
    dstp = jnp.concatenate([dst, pad])

    deg_k, mp_k = _sc_kernels()
    degp = deg_k(dstp).reshape(NC, NP, 1)
    y1, dinv = _tc_pre(x, W1, degp)
    p1 = mp_k(y1, srcp, dstp)
    h1, y2 = _tc_mid(p1, y1, dinv, b1[None, :], gamma[None, :], beta[None, :], W2)
    p2 = mp_k(y2, srcp, dstp)
    ids1p, c1 = _tc_vq1_call(h1, cb1)

    dout = Wg.shape[1]
    _tc_fin = pl.pallas_call(
        _tc_fin_body,
        grid=(VG,),
        in_specs=[
            pl.BlockSpec((2, VB, D), lambda i: (0, i, 0)),
            pl.BlockSpec((VB, D), lambda i: (i, 0)),
            pl.BlockSpec((VB, 1), lambda i: (i, 0)),
            _full((1, D)),
            _full((R, K, D)),
            _full((D, D)),
            _full((1, D)),
            _full((D, dout)),
            _full((1, dout)),
            _full((1, 1)),
        ],
        out_specs=[
            pl.BlockSpec((VB, D), lambda i: (i, 0)),
            pl.BlockSpec((VB, dout), lambda i: (i, 0)),
            pl.BlockSpec((VB, 1), lambda i: (i, 0)),
            _full((1, 1)),
        ],
        out_shape=[
            jax.ShapeDtypeStruct((N, D), jnp.float32),
            jax.ShapeDtypeStruct((N, dout), jnp.float32),
            jax.ShapeDtypeStruct((N, 1), jnp.int32),
            jax.ShapeDtypeStruct((1, 1), jnp.float32),
        ],
    )
    o1, og, ids2p, cm = _tc_fin(p2, y2, dinv, b2[None, :], cb2, Wl, bl[None, :],
                                Wg, bg[None, :], c1)

    ids = []
    for packed in (ids1p, ids2p):
        for l in range(R):
            ids.append((packed >> (5 * l)) & 31)
    id_concat = jnp.concatenate(ids, axis=1)
    return (o1, cm[0, 0], id_concat, og)


# speedup vs baseline: 8.1983x; 1.2036x over previous
import functools

import jax
import jax.numpy as jnp
from jax import lax
from jax.experimental import pallas as pl
from jax.experimental.pallas import tpu as pltpu
from jax.experimental.pallas import tpu_sc as plsc

N = 10000
D = 128
E = 320000
K = 32
R = 3
EPS = 1e-8

NC = 2
NS = 16
NW = NC * NS
NP = 10240
ROWS_PER_TILE = NP // NS
C = 128
T = 80
EP = NW * T * C
DW = 16


def _deg_body(dst_hbm, out_hbm, didx_v, ones_v, stage_v, acc_sh):
    c = lax.axis_index("c")
    s = lax.axis_index("s")
    wid = s * NC + c

    def fill(i, _):
        ones_v[pl.ds(i * 16, 16)] = jnp.full((16,), 1.0, jnp.float32)
        return 0

    lax.fori_loop(0, C // 16, fill, 0)

    def zfill(i, _):
        stage_v[pl.ds(i * 16, 16)] = jnp.zeros((16,), jnp.float32)
        return 0

    lax.fori_loop(0, ROWS_PER_TILE // 16, zfill, 0)

    base = s * ROWS_PER_TILE
    pltpu.sync_copy(stage_v, acc_sh.at[pl.ds(base, ROWS_PER_TILE)])
    plsc.subcore_barrier()

    def body(j, _):
        pltpu.sync_copy(dst_hbm.at[pl.ds(wid * T * C + j * C, C)], didx_v)
        pltpu.sync_copy(ones_v, acc_sh.at[didx_v], add=True)
        return 0

    lax.fori_loop(0, T, body, 0)
    plsc.subcore_barrier()

    pltpu.sync_copy(acc_sh.at[pl.ds(base, ROWS_PER_TILE)], stage_v)
    pltpu.sync_copy(stage_v, out_hbm.at[c, pl.ds(base, ROWS_PER_TILE)])


NBUF = 2


def _mp_body(y_hbm, src_hbm, dst_hbm, out_hbm, sidx_v, didx_v, rows_v, acc_sh,
             sems):
    c = lax.axis_index("c")
    s = lax.axis_index("s")
    wid = s * NC + c

    def zero_rows(i, _):
        r = i // (D // 16)
        q = i % (D // 16)
        rows_v[0][r, pl.ds(q * 16, 16)] = jnp.zeros((16,), jnp.float32)
        return 0

    lax.fori_loop(0, C * (D // 16), zero_rows, 0)

    base = s * ROWS_PER_TILE
    for k in range(ROWS_PER_TILE // C):
        pltpu.sync_copy(rows_v[0], acc_sh.at[pl.ds(base + k * C, C)])
    plsc.subcore_barrier()

    pltpu.sync_copy(src_hbm.at[pl.ds(wid * T, T)], sidx_v)

    for b in range(NBUF):
        pltpu.async_copy(y_hbm.at[sidx_v.at[b]], rows_v[b], sems[b])

    def body(i, _):
        for bb in range(NBUF):
            g = i * NBUF + bb
            pltpu.make_async_copy(y_hbm.at[sidx_v.at[g]], rows_v[bb],
                                  sems[bb]).wait()
            pltpu.sync_copy(dst_hbm.at[pl.ds(wid * T * C + g * C, C)], didx_v)
            pltpu.sync_copy(rows_v[bb], acc_sh.at[didx_v], add=True)

            @pl.when(g + NBUF < T)
            def _():
                pltpu.async_copy(y_hbm.at[sidx_v.at[g + NBUF]], rows_v[bb],
                                 sems[bb])

        return 0

    lax.fori_loop(0, T // NBUF, body, 0)
    plsc.subcore_barrier()

    for k in range(ROWS_PER_TILE // C):
        pltpu.sync_copy(acc_sh.at[pl.ds(base + k * C, C)], rows_v[0])
        pltpu.sync_copy(rows_v[0], out_hbm.at[c, pl.ds(base + k * C, C)])


def _tc_pre_body(x_ref, w1_ref, degp_ref, y1_ref, dinv_ref):
    dp = degp_ref[...]
    deg = dp[0] + dp[1] + 1.0
    dinv = lax.rsqrt(jnp.maximum(deg, 1e-12))
    dinv_ref[...] = dinv
    xw = jnp.dot(x_ref[...], w1_ref[...], preferred_element_type=jnp.float32)
    y1_ref[0:N, :] = dinv[0:N] * xw
    y1_ref[N:NP, :] = jnp.zeros((NP - N, D), jnp.float32)


def _tc_mid_body(p_ref, y1_ref, dinv_ref, b1_ref, gamma_ref, beta_ref, w2_ref,
                 h1_ref, y2_ref):
    p = p_ref[...]
    dinv = dinv_ref[0:N]
    h = dinv * (p[0, 0:N] + p[1, 0:N] + y1_ref[0:N]) + b1_ref[...]
    mu = jnp.mean(h, axis=0, keepdims=True)
    var = jnp.mean((h - mu) ** 2, axis=0, keepdims=True)
    h = gamma_ref[...] * (h - mu) / jnp.sqrt(var + 1e-5) + beta_ref[...]
    h = jnp.maximum(h, 0.0)
    h1_ref[...] = h
    xw2 = jnp.dot(h, w2_ref[...], preferred_element_type=jnp.float32)
    y2_ref[0:N, :] = dinv * xw2
    y2_ref[N:NP, :] = jnp.zeros((NP - N, D), jnp.float32)


VB = 2000
VG = N // VB


def _vq(h, cb_all):
    residual = h
    b = h.shape[0]
    commit = jnp.float32(0.0)
    packed = jnp.zeros((b, 1), jnp.int32)
    for l in range(R):
        cb = cb_all[l]
        rn = residual / (jnp.sqrt(jnp.sum(residual * residual, axis=-1,
                                          keepdims=True)) + EPS)
        cbn = cb / (jnp.sqrt(jnp.sum(cb * cb, axis=-1, keepdims=True)) + EPS)
        sim = lax.dot_general(rn, cbn, (((1,), (1,)), ((), ())),
                              preferred_element_type=jnp.float32)
        mx = jnp.max(sim, axis=-1, keepdims=True)
        lane = lax.broadcasted_iota(jnp.int32, (b, K), 1)
        idx = jnp.min(jnp.where(sim >= mx, lane, K), axis=-1, keepdims=True)
        q = jnp.zeros_like(residual)
        for k in range(K):
            q = jnp.where(idx == k, cb[k][None, :], q)
        commit = commit + 0.25 * (jnp.sum((q - residual) ** 2) / (N * D))
        residual = residual - q
        packed = packed + (idx << (5 * l))
    return packed, commit


def _tc_vq1_body(h1_ref, cb1_ref, ids_ref, c1_ref):
    i = pl.program_id(0)
    packed, commit = _vq(h1_ref[...], cb1_ref[...])
    ids_ref[...] = packed

    @pl.when(i == 0)
    def _():
        c1_ref[...] = jnp.zeros((1, 1), jnp.float32)

    c1_ref[...] += jnp.reshape(commit, (1, 1))


def _tc_fin_body(p_ref, y2_ref, dinv_ref, b2_ref, cb2_ref, wl_ref, bl_ref,
                 wg_ref, bg_ref, c1_ref, o1_ref, og_ref, ids_ref, cm_ref):
    i = pl.program_id(0)
    p = p_ref[...]
    h = dinv_ref[...] * (p[0] + p[1] + y2_ref[...]) + b2_ref[...]
    packed, c2 = _vq(h, cb2_ref[...])
    ids_ref[...] = packed

    @pl.when(i == 0)
    def _():
        cm_ref[...] = c1_ref[...]

    cm_ref[...] += jnp.reshape(c2, (1, 1))
    o1_ref[...] = jnp.dot(h, wl_ref[...], preferred_element_type=jnp.float32) \
        + bl_ref[...]
    og_ref[...] = jnp.dot(h, wg_ref[...], preferred_element_type=jnp.float32) \
        + bg_ref[...]


def _mk(body, out_shapes):
    return pl.pallas_call(body, out_shape=out_shapes)


_full = lambda shape: pl.BlockSpec(shape, lambda i: tuple(0 for _ in shape))

_tc_vq1_call = pl.pallas_call(
    _tc_vq1_body,
    grid=(VG,),
    in_specs=[
        pl.BlockSpec((VB, D), lambda i: (i, 0)),
        _full((R, K, D)),
    ],
    out_specs=[
        pl.BlockSpec((VB, 1), lambda i: (i, 0)),
        _full((1, 1)),
    ],
    out_shape=[
        jax.ShapeDtypeStruct((N, 1), jnp.int32),
        jax.ShapeDtypeStruct((1, 1), jnp.float32),
    ],
)


@functools.cache
def _sc_kernels():
    mesh = plsc.VectorSubcoreMesh(core_axis_name="c", subcore_axis_name="s",
                                  num_cores=NC, num_subcores=NS)
    deg = pl.kernel(
        _deg_body,
        out_type=jax.ShapeDtypeStruct((NC, NP), jnp.float32),
        mesh=mesh,
        scratch_types=[
            pltpu.VMEM((C,), jnp.int32),
            pltpu.VMEM((C,), jnp.float32),
            pltpu.VMEM((ROWS_PER_TILE,), jnp.float32),
            pltpu.VMEM_SHARED((NP,), jnp.float32),
        ],
    )
    mp = pl.kernel(
        _mp_body,
        out_type=jax.ShapeDtypeStruct((NC, NP, D), jnp.float32),
        mesh=mesh,
        scratch_types=[
            pltpu.VMEM((T, C), jnp.int32),
            pltpu.VMEM((C,), jnp.int32),
            [pltpu.VMEM((C, D), jnp.float32)] * NBUF,
            pltpu.VMEM_SHARED((NP, D), jnp.float32),
            [pltpu.SemaphoreType.DMA] * NBUF,
        ],
    )
    return deg, mp


_tc_pre = _mk(_tc_pre_body, [
    jax.ShapeDtypeStruct((NP, D), jnp.float32),
    jax.ShapeDtypeStruct((NP, 1), jnp.float32),
])
_tc_mid = _mk(_tc_mid_body, [
    jax.ShapeDtypeStruct((N, D), jnp.float32),
    jax.ShapeDtypeStruct((NP, D), jnp.float32),
])


def kernel(x, edge_index, W1, b1, W2, b2, gamma, beta, cb1, cb2, Wl, bl, Wg, bg):
    src = edge_index[0]
    dst = edge_index[1]
    pad = jnp.full((EP - E,), NP - 1, dtype=jnp.int32)
    srcp = jnp.concatenate([src, pad])
    dstp = jnp.concatenate([dst, pad])
    src2d = srcp.reshape(NW * T, C)

    deg_k, mp_k = _sc_kernels()
    degp = deg_k(dstp).reshape(NC, NP, 1)
    y1, dinv = _tc_pre(x, W1, degp)
    p1 = mp_k(y1, src2d, dstp)
    h1, y2 = _tc_mid(p1, y1, dinv, b1[None, :], gamma[None, :], beta[None, :], W2)
    p2 = mp_k(y2, src2d, dstp)
    ids1p, c1 = _tc_vq1_call(h1, cb1)

    dout = Wg.shape[1]
    _tc_fin = pl.pallas_call(
        _tc_fin_body,
        grid=(VG,),
        in_specs=[
            pl.BlockSpec((2, VB, D), lambda i: (0, i, 0)),
            pl.BlockSpec((VB, D), lambda i: (i, 0)),
            pl.BlockSpec((VB, 1), lambda i: (i, 0)),
            _full((1, D)),
            _full((R, K, D)),
            _full((D, D)),
            _full((1, D)),
            _full((D, dout)),
            _full((1, dout)),
            _full((1, 1)),
        ],
        out_specs=[
            pl.BlockSpec((VB, D), lambda i: (i, 0)),
            pl.BlockSpec((VB, dout), lambda i: (i, 0)),
            pl.BlockSpec((VB, 1), lambda i: (i, 0)),
            _full((1, 1)),
        ],
        out_shape=[
            jax.ShapeDtypeStruct((N, D), jnp.float32),
            jax.ShapeDtypeStruct((N, dout), jnp.float32),
            jax.ShapeDtypeStruct((N, 1), jnp.int32),
            jax.ShapeDtypeStruct((1, 1), jnp.float32),
        ],
    )
    o1, og, ids2p, cm = _tc_fin(p2, y2, dinv, b2[None, :], cb2, Wl, bl[None, :],
                                Wg, bg[None, :], c1)

    ids = []
    for packed in (ids1p, ids2p):
        for l in range(R):
            ids.append((packed >> (5 * l)) & 31)
    id_concat = jnp.concatenate(ids, axis=1)
    return (o1, cm[0, 0], id_concat, og)


# speedup vs baseline: 8.6917x; 1.0602x over previous
import functools

import jax
import jax.numpy as jnp
from jax import lax
from jax.experimental import pallas as pl
from jax.experimental.pallas import tpu as pltpu
from jax.experimental.pallas import tpu_sc as plsc

N = 10000
D = 128
E = 320000
K = 32
R = 3
EPS = 1e-8

NC = 2
NS = 16
NW = NC * NS
NP = 10240
ROWS_PER_TILE = NP // NS
C = 128
T = 80
EP = NW * T * C
DW = 16


def _deg_body(dst_hbm, out_hbm, didx_v, ones_v, stage_v, acc_sh):
    c = lax.axis_index("c")
    s = lax.axis_index("s")
    wid = s * NC + c

    def fill(i, _):
        ones_v[pl.ds(i * 16, 16)] = jnp.full((16,), 1.0, jnp.float32)
        return 0

    lax.fori_loop(0, C // 16, fill, 0)

    def zfill(i, _):
        stage_v[pl.ds(i * 16, 16)] = jnp.zeros((16,), jnp.float32)
        return 0

    lax.fori_loop(0, ROWS_PER_TILE // 16, zfill, 0)

    base = s * ROWS_PER_TILE
    pltpu.sync_copy(stage_v, acc_sh.at[pl.ds(base, ROWS_PER_TILE)])
    plsc.subcore_barrier()

    def body(j, _):
        pltpu.sync_copy(dst_hbm.at[pl.ds(wid * T * C + j * C, C)], didx_v)
        pltpu.sync_copy(ones_v, acc_sh.at[didx_v], add=True)
        return 0

    lax.fori_loop(0, T, body, 0)
    plsc.subcore_barrier()

    pltpu.sync_copy(acc_sh.at[pl.ds(base, ROWS_PER_TILE)], stage_v)
    pltpu.sync_copy(stage_v, out_hbm.at[c, pl.ds(base, ROWS_PER_TILE)])


NBUF = 2


def _mp_body(pk_hbm, y_hbm, out_hbm, pidx_v, sidx_c, didx_v, rows_v, acc_sh,
             sems):
    c = lax.axis_index("c")
    s = lax.axis_index("s")
    wid = s * NC + c

    def zero_rows(i, _):
        r = i // (D // 16)
        q = i % (D // 16)
        rows_v[0][r, pl.ds(q * 16, 16)] = jnp.zeros((16,), jnp.float32)
        return 0

    lax.fori_loop(0, C * (D // 16), zero_rows, 0)

    base = s * ROWS_PER_TILE
    for k in range(ROWS_PER_TILE // C):
        pltpu.sync_copy(rows_v[0], acc_sh.at[pl.ds(base + k * C, C)])
    plsc.subcore_barrier()

    pltpu.sync_copy(pk_hbm.at[pl.ds(wid * T, T)], pidx_v)

    def unpack_src(g, bb):
        for q in range(C // 16):
            v = pidx_v[g, pl.ds(q * 16, 16)]
            sidx_c[bb][pl.ds(q * 16, 16)] = v & 16383

    def unpack_dst(g):
        for q in range(C // 16):
            v = pidx_v[g, pl.ds(q * 16, 16)]
            didx_v[pl.ds(q * 16, 16)] = lax.shift_right_logical(v, 14)

    for b in range(NBUF):
        unpack_src(b, b)
        pltpu.async_copy(y_hbm.at[sidx_c[b]], rows_v[b], sems[b])

    def body(i, _):
        for bb in range(NBUF):
            g = i * NBUF + bb
            pltpu.make_async_copy(y_hbm.at[sidx_c[bb]], rows_v[bb],
                                  sems[bb]).wait()
            unpack_dst(g)
            pltpu.sync_copy(rows_v[bb], acc_sh.at[didx_v], add=True)

            @pl.when(g + NBUF < T)
            def _():
                unpack_src(g + NBUF, bb)
                pltpu.async_copy(y_hbm.at[sidx_c[bb]], rows_v[bb], sems[bb])

        return 0

    lax.fori_loop(0, T // NBUF, body, 0)
    plsc.subcore_barrier()

    for k in range(ROWS_PER_TILE // C):
        pltpu.sync_copy(acc_sh.at[pl.ds(base + k * C, C)], rows_v[0])
        pltpu.sync_copy(rows_v[0], out_hbm.at[c, pl.ds(base + k * C, C)])


def _tc_pre_body(x_ref, w1_ref, degp_ref, y1_ref, dinv_ref):
    dp = degp_ref[...]
    deg = dp[0] + dp[1] + 1.0
    dinv = lax.rsqrt(jnp.maximum(deg, 1e-12))
    dinv_ref[...] = dinv
    xw = jnp.dot(x_ref[...], w1_ref[...], preferred_element_type=jnp.float32)
    y1_ref[0:N, :] = dinv[0:N] * xw
    y1_ref[N:NP, :] = jnp.zeros((NP - N, D), jnp.float32)


def _tc_mid_body(p_ref, y1_ref, dinv_ref, b1_ref, gamma_ref, beta_ref, w2_ref,
                 h1_ref, y2_ref):
    p = p_ref[...]
    dinv = dinv_ref[0:N]
    h = dinv * (p[0, 0:N] + p[1, 0:N] + y1_ref[0:N]) + b1_ref[...]
    mu = jnp.mean(h, axis=0, keepdims=True)
    var = jnp.mean((h - mu) ** 2, axis=0, keepdims=True)
    h = gamma_ref[...] * (h - mu) / jnp.sqrt(var + 1e-5) + beta_ref[...]
    h = jnp.maximum(h, 0.0)
    h1_ref[...] = h
    xw2 = jnp.dot(h, w2_ref[...], preferred_element_type=jnp.float32)
    y2_ref[0:N, :] = dinv * xw2
    y2_ref[N:NP, :] = jnp.zeros((NP - N, D), jnp.float32)


VB = 2000
VG = N // VB


def _vq(h, cb_all):
    residual = h
    b = h.shape[0]
    commit = jnp.float32(0.0)
    packed = jnp.zeros((b, 1), jnp.int32)
    for l in range(R):
        cb = cb_all[l]
        rn = residual / (jnp.sqrt(jnp.sum(residual * residual, axis=-1,
                                          keepdims=True)) + EPS)
        cbn = cb / (jnp.sqrt(jnp.sum(cb * cb, axis=-1, keepdims=True)) + EPS)
        sim = lax.dot_general(rn, cbn, (((1,), (1,)), ((), ())),
                              preferred_element_type=jnp.float32)
        mx = jnp.max(sim, axis=-1, keepdims=True)
        lane = lax.broadcasted_iota(jnp.int32, (b, K), 1)
        idx = jnp.min(jnp.where(sim >= mx, lane, K), axis=-1, keepdims=True)
        q = jnp.zeros_like(residual)
        for k in range(K):
            q = jnp.where(idx == k, cb[k][None, :], q)
        commit = commit + 0.25 * (jnp.sum((q - residual) ** 2) / (N * D))
        residual = residual - q
        packed = packed + (idx << (5 * l))
    return packed, commit


def _tc_vq1_body(h1_ref, cb1_ref, ids_ref, c1_ref):
    i = pl.program_id(0)
    packed, commit = _vq(h1_ref[...], cb1_ref[...])
    ids_ref[...] = packed

    @pl.when(i == 0)
    def _():
        c1_ref[...] = jnp.zeros((1, 1), jnp.float32)

    c1_ref[...] += jnp.reshape(commit, (1, 1))


def _tc_fin_body(p_ref, y2_ref, dinv_ref, b2_ref, cb2_ref, wl_ref, bl_ref,
                 wg_ref, bg_ref, c1_ref, o1_ref, og_ref, ids_ref, cm_ref):
    i = pl.program_id(0)
    p = p_ref[...]
    h = dinv_ref[...] * (p[0] + p[1] + y2_ref[...]) + b2_ref[...]
    packed, c2 = _vq(h, cb2_ref[...])
    ids_ref[...] = packed

    @pl.when(i == 0)
    def _():
        cm_ref[...] = c1_ref[...]

    cm_ref[...] += jnp.reshape(c2, (1, 1))
    o1_ref[...] = jnp.dot(h, wl_ref[...], preferred_element_type=jnp.float32) \
        + bl_ref[...]
    og_ref[...] = jnp.dot(h, wg_ref[...], preferred_element_type=jnp.float32) \
        + bg_ref[...]


def _mk(body, out_shapes):
    return pl.pallas_call(body, out_shape=out_shapes)


_full = lambda shape: pl.BlockSpec(shape, lambda i: tuple(0 for _ in shape))

_tc_vq1_call = pl.pallas_call(
    _tc_vq1_body,
    grid=(VG,),
    in_specs=[
        pl.BlockSpec((VB, D), lambda i: (i, 0)),
        _full((R, K, D)),
    ],
    out_specs=[
        pl.BlockSpec((VB, 1), lambda i: (i, 0)),
        _full((1, 1)),
    ],
    out_shape=[
        jax.ShapeDtypeStruct((N, 1), jnp.int32),
        jax.ShapeDtypeStruct((1, 1), jnp.float32),
    ],
)


@functools.cache
def _sc_kernels():
    mesh = plsc.VectorSubcoreMesh(core_axis_name="c", subcore_axis_name="s",
                                  num_cores=NC, num_subcores=NS)
    deg = pl.kernel(
        _deg_body,
        out_type=jax.ShapeDtypeStruct((NC, NP), jnp.float32),
        mesh=mesh,
        scratch_types=[
            pltpu.VMEM((C,), jnp.int32),
            pltpu.VMEM((C,), jnp.float32),
            pltpu.VMEM((ROWS_PER_TILE,), jnp.float32),
            pltpu.VMEM_SHARED((NP,), jnp.float32),
        ],
    )
    mp = pl.kernel(
        _mp_body,
        out_type=jax.ShapeDtypeStruct((NC, NP, D), jnp.float32),
        mesh=mesh,
        scratch_types=[
            pltpu.VMEM((T, C), jnp.int32),
            [pltpu.VMEM((C,), jnp.int32)] * NBUF,
            pltpu.VMEM((C,), jnp.int32),
            [pltpu.VMEM((C, D), jnp.float32)] * NBUF,
            pltpu.VMEM_SHARED((NP, D), jnp.float32),
            [pltpu.SemaphoreType.DMA] * NBUF,
        ],
    )
    return deg, mp


_tc_pre = _mk(_tc_pre_body, [
    jax.ShapeDtypeStruct((NP, D), jnp.float32),
    jax.ShapeDtypeStruct((NP, 1), jnp.float32),
])
_tc_mid = _mk(_tc_mid_body, [
    jax.ShapeDtypeStruct((N, D), jnp.float32),
    jax.ShapeDtypeStruct((NP, D), jnp.float32),
])


def kernel(x, edge_index, W1, b1, W2, b2, gamma, beta, cb1, cb2, Wl, bl, Wg, bg):
    src = edge_index[0]
    dst = edge_index[1]
    pad = jnp.full((EP - E,), NP - 1, dtype=jnp.int32)
    srcp = jnp.concatenate([src, pad])
    dstp = jnp.concatenate([dst, pad])
    pk2d = (srcp + dstp * 16384).reshape(NW * T, C)

    deg_k, mp_k = _sc_kernels()
    degp = deg_k(dstp).reshape(NC, NP, 1)
    y1, dinv = _tc_pre(x, W1, degp)
    p1 = mp_k(pk2d, y1)
    h1, y2 = _tc_mid(p1, y1, dinv, b1[None, :], gamma[None, :], beta[None, :], W2)
    p2 = mp_k(pk2d, y2)
    ids1p, c1 = _tc_vq1_call(h1, cb1)

    dout = Wg.shape[1]
    _tc_fin = pl.pallas_call(
        _tc_fin_body,
        grid=(VG,),
        in_specs=[
            pl.BlockSpec((2, VB, D), lambda i: (0, i, 0)),
            pl.BlockSpec((VB, D), lambda i: (i, 0)),
            pl.BlockSpec((VB, 1), lambda i: (i, 0)),
            _full((1, D)),
            _full((R, K, D)),
            _full((D, D)),
            _full((1, D)),
            _full((D, dout)),
            _full((1, dout)),
            _full((1, 1)),
        ],
        out_specs=[
            pl.BlockSpec((VB, D), lambda i: (i, 0)),
            pl.BlockSpec((VB, dout), lambda i: (i, 0)),
            pl.BlockSpec((VB, 1), lambda i: (i, 0)),
            _full((1, 1)),
        ],
        out_shape=[
            jax.ShapeDtypeStruct((N, D), jnp.float32),
            jax.ShapeDtypeStruct((N, dout), jnp.float32),
            jax.ShapeDtypeStruct((N, 1), jnp.int32),
            jax.ShapeDtypeStruct((1, 1), jnp.float32),
        ],
    )
    o1, og, ids2p, cm = _tc_fin(p2, y2, dinv, b2[None, :], cb2, Wl, bl[None, :],
                                Wg, bg[None, :], c1)

    ids = []
    for packed in (ids1p, ids2p):
        for l in range(R):
            ids.append((packed >> (5 * l)) & 31)
    id_concat = jnp.concatenate(ids, axis=1)
    return (o1, cm[0, 0], id_concat, og)
